# Initial kernel scaffold; baseline (speedup 1.0000x reference)
import functools

import jax
import jax.numpy as jnp
from jax import lax
from jax.experimental import pallas as pl
from jax.experimental.pallas import tpu as pltpu
from jax.experimental.pallas import tpu_sc as plsc

VOCAB_N = 1000000
EMBED = 128
HIDDEN = 128
OUT_N = 91
BATCH_N = 16384
CTX_N = 6

_info = plsc.get_sparse_core_info()
NC, NS = _info.num_cores, _info.num_subcores
NW = NC * NS

TOTAL_ROWS = BATCH_N * CTX_N
ROWS_PER_W = TOTAL_ROWS // NW
CHUNK = 384
NCHUNK = ROWS_PER_W // CHUNK


def _make_gather():
    mesh = plsc.VectorSubcoreMesh(core_axis_name="c", subcore_axis_name="s")

    @functools.partial(
        pl.kernel,
        mesh=mesh,
        out_type=jax.ShapeDtypeStruct((TOTAL_ROWS, EMBED), jnp.float32),
        scratch_types=[
            pltpu.VMEM((ROWS_PER_W,), jnp.int32),
            pltpu.VMEM((CHUNK, EMBED), jnp.float32),
            pltpu.VMEM((CHUNK, EMBED), jnp.float32),
            pltpu.SemaphoreType.DMA,
            pltpu.SemaphoreType.DMA,
        ],
    )
    def gather(table_hbm, idx_hbm, out_hbm, idx_v, rows0, rows1, sem0, sem1):
        wid = lax.axis_index("s") * NC + lax.axis_index("c")
        base = wid * ROWS_PER_W
        pltpu.sync_copy(idx_hbm.at[pl.ds(base, ROWS_PER_W)], idx_v)
        bufs = (rows0, rows1)
        sems = (sem0, sem1)
        pltpu.async_copy(
            table_hbm.at[idx_v.at[pl.ds(0, CHUNK)]], bufs[0], sems[0])
        for c in range(NCHUNK):
            buf, sem = bufs[c % 2], sems[c % 2]
            pltpu.make_async_copy(
                table_hbm.at[idx_v.at[pl.ds(c * CHUNK, CHUNK)]], buf, sem
            ).wait()
            if c + 1 < NCHUNK:
                pltpu.async_copy(
                    table_hbm.at[idx_v.at[pl.ds((c + 1) * CHUNK, CHUNK)]],
                    bufs[(c + 1) % 2], sems[(c + 1) % 2])
            pltpu.sync_copy(buf, out_hbm.at[pl.ds(base + c * CHUNK, CHUNK)])

    return gather


_gather = _make_gather()


def _mlp_body(x_ref, w1_ref, b1_ref, w2_ref, b2_ref, out_ref):
    h = jnp.dot(x_ref[...], w1_ref[...], preferred_element_type=jnp.float32)
    h = jnp.maximum(h + b1_ref[...], 0.0)
    out_ref[...] = (
        jnp.dot(h, w2_ref[...], preferred_element_type=jnp.float32)
        + b2_ref[...]
    )


BM = 1024


def _mlp(x, W1, b1, W2, b2):
    grid = (BATCH_N // BM,)
    return pl.pallas_call(
        _mlp_body,
        grid=grid,
        in_specs=[
            pl.BlockSpec((BM, CTX_N * EMBED), lambda i: (i, 0)),
            pl.BlockSpec((CTX_N * EMBED, HIDDEN), lambda i: (0, 0)),
            pl.BlockSpec((1, HIDDEN), lambda i: (0, 0)),
            pl.BlockSpec((HIDDEN, OUT_N), lambda i: (0, 0)),
            pl.BlockSpec((1, OUT_N), lambda i: (0, 0)),
        ],
        out_specs=pl.BlockSpec((BM, OUT_N), lambda i: (i, 0)),
        out_shape=jax.ShapeDtypeStruct((BATCH_N, OUT_N), jnp.float32),
    )(x, W1, b1, W2, b2)


def kernel(inputs, table, W1, b1, W2, b2):
    flat_idx = inputs.reshape(-1)
    embeds = _gather(table, flat_idx)
    x = embeds.reshape(BATCH_N, CTX_N * EMBED)
    return _mlp(x, W1, b1.reshape(1, HIDDEN), W2, b2.reshape(1, OUT_N))


# speedup vs baseline: 11.6144x; 11.6144x over previous
import functools

import jax
import jax.numpy as jnp
from jax import lax
from jax.experimental import pallas as pl
from jax.experimental.pallas import tpu as pltpu
from jax.experimental.pallas import tpu_sc as plsc

VOCAB_N = 1000000
EMBED = 128
HIDDEN = 128
OUT_N = 91
BATCH_N = 16384
CTX_N = 6

NC, NS = 2, 16
NW = NC * NS

TOTAL_ROWS = BATCH_N * CTX_N
ROWS_PER_W = TOTAL_ROWS // NW
CHUNK = 384
NCHUNK = ROWS_PER_W // CHUNK


def _make_gather():
    mesh = plsc.VectorSubcoreMesh(core_axis_name="c", subcore_axis_name="s")

    @functools.partial(
        pl.kernel,
        mesh=mesh,
        out_type=jax.ShapeDtypeStruct((TOTAL_ROWS, EMBED), jnp.float32),
        scratch_types=[
            pltpu.VMEM((ROWS_PER_W,), jnp.int32),
            pltpu.VMEM((CHUNK, EMBED), jnp.float32),
            pltpu.VMEM((CHUNK, EMBED), jnp.float32),
            pltpu.SemaphoreType.DMA,
            pltpu.SemaphoreType.DMA,
        ],
    )
    def gather(table_hbm, idx_hbm, out_hbm, idx_v, rows0, rows1, sem0, sem1):
        wid = lax.axis_index("s") * NC + lax.axis_index("c")
        base = wid * ROWS_PER_W
        pltpu.sync_copy(idx_hbm.at[pl.ds(base, ROWS_PER_W)], idx_v)
        bufs = (rows0, rows1)
        sems = (sem0, sem1)
        pltpu.async_copy(
            table_hbm.at[idx_v.at[pl.ds(0, CHUNK)]], bufs[0], sems[0])
        for c in range(NCHUNK):
            buf, sem = bufs[c % 2], sems[c % 2]
            pltpu.make_async_copy(
                table_hbm.at[idx_v.at[pl.ds(c * CHUNK, CHUNK)]], buf, sem
            ).wait()
            if c + 1 < NCHUNK:
                pltpu.async_copy(
                    table_hbm.at[idx_v.at[pl.ds((c + 1) * CHUNK, CHUNK)]],
                    bufs[(c + 1) % 2], sems[(c + 1) % 2])
            pltpu.sync_copy(buf, out_hbm.at[pl.ds(base + c * CHUNK, CHUNK)])

    return gather


_gather = _make_gather()


def _mlp_body(x_ref, w1_ref, b1_ref, w2_ref, b2_ref, out_ref):
    h = jnp.dot(x_ref[...], w1_ref[...], preferred_element_type=jnp.float32)
    h = jnp.maximum(h + b1_ref[...], 0.0)
    out_ref[...] = (
        jnp.dot(h, w2_ref[...], preferred_element_type=jnp.float32)
        + b2_ref[...]
    )


BM = 1024


def _mlp(x, W1, b1, W2, b2):
    grid = (BATCH_N // BM,)
    return pl.pallas_call(
        _mlp_body,
        grid=grid,
        in_specs=[
            pl.BlockSpec((BM, CTX_N * EMBED), lambda i: (i, 0)),
            pl.BlockSpec((CTX_N * EMBED, HIDDEN), lambda i: (0, 0)),
            pl.BlockSpec((1, HIDDEN), lambda i: (0, 0)),
            pl.BlockSpec((HIDDEN, OUT_N), lambda i: (0, 0)),
            pl.BlockSpec((1, OUT_N), lambda i: (0, 0)),
        ],
        out_specs=pl.BlockSpec((BM, OUT_N), lambda i: (i, 0)),
        out_shape=jax.ShapeDtypeStruct((BATCH_N, OUT_N), jnp.float32),
    )(x, W1, b1, W2, b2)


def kernel(inputs, table, W1, b1, W2, b2):
    flat_idx = inputs.reshape(-1)
    embeds = _gather(table, flat_idx)
    x = embeds.reshape(BATCH_N, CTX_N * EMBED)
    return _mlp(x, W1, b1.reshape(1, HIDDEN), W2, b2.reshape(1, OUT_N))


# speedup vs baseline: 20.0529x; 1.7265x over previous
import functools

import jax
import jax.numpy as jnp
from jax import lax
from jax.experimental import pallas as pl
from jax.experimental.pallas import tpu as pltpu
from jax.experimental.pallas import tpu_sc as plsc

VOCAB_N = 1000000
EMBED = 128
HIDDEN = 128
OUT_N = 91
BATCH_N = 16384
CTX_N = 6

NC, NS = 2, 16
NW = NC * NS

TOTAL_ROWS = BATCH_N * CTX_N
ROWS_PER_W = TOTAL_ROWS // NW
CHUNK = 384
NCHUNK = ROWS_PER_W // CHUNK


def _make_gather():
    mesh = plsc.VectorSubcoreMesh(core_axis_name="c", subcore_axis_name="s")

    @functools.partial(
        pl.kernel,
        mesh=mesh,
        out_type=jax.ShapeDtypeStruct((TOTAL_ROWS, EMBED), jnp.float32),
        scratch_types=[
            pltpu.VMEM((ROWS_PER_W,), jnp.int32),
            pltpu.VMEM((CHUNK, EMBED), jnp.float32),
            pltpu.VMEM((CHUNK, EMBED), jnp.float32),
            pltpu.SemaphoreType.DMA,
            pltpu.SemaphoreType.DMA,
        ],
    )
    def gather(table_hbm, idx_hbm, out_hbm, idx_v, rows0, rows1, sem0, sem1):
        wid = lax.axis_index("s") * NC + lax.axis_index("c")
        base = wid * ROWS_PER_W
        pltpu.sync_copy(idx_hbm.at[pl.ds(base, ROWS_PER_W)], idx_v)
        bufs = (rows0, rows1)
        sems = (sem0, sem1)
        pltpu.async_copy(
            table_hbm.at[idx_v.at[pl.ds(0, CHUNK)]], bufs[0], sems[0])
        for c in range(NCHUNK):
            buf, sem = bufs[c % 2], sems[c % 2]
            pltpu.make_async_copy(
                table_hbm.at[idx_v.at[pl.ds(c * CHUNK, CHUNK)]], buf, sem
            ).wait()
            if c + 1 < NCHUNK:
                pltpu.async_copy(
                    table_hbm.at[idx_v.at[pl.ds((c + 1) * CHUNK, CHUNK)]],
                    bufs[(c + 1) % 2], sems[(c + 1) % 2])
            pltpu.sync_copy(buf, out_hbm.at[pl.ds(base + c * CHUNK, CHUNK)])

    return gather


_gather = _make_gather()


def _mlp_body(x_ref, w1_ref, b1_ref, w2_ref, b2_ref, out_ref):
    h = b1_ref[...]
    for c in range(CTX_N):
        h = h + jnp.dot(x_ref[c], w1_ref[c],
                        preferred_element_type=jnp.float32)
    h = jnp.maximum(h, 0.0)
    out_ref[...] = (
        jnp.dot(h, w2_ref[...], preferred_element_type=jnp.float32)
        + b2_ref[...]
    )


BM = 1024


def _mlp(x3, W1r, b1, W2, b2):
    grid = (BATCH_N // BM,)
    return pl.pallas_call(
        _mlp_body,
        grid=grid,
        in_specs=[
            pl.BlockSpec((CTX_N, BM, EMBED), lambda i: (0, i, 0)),
            pl.BlockSpec((CTX_N, EMBED, HIDDEN), lambda i: (0, 0, 0)),
            pl.BlockSpec((1, HIDDEN), lambda i: (0, 0)),
            pl.BlockSpec((HIDDEN, OUT_N), lambda i: (0, 0)),
            pl.BlockSpec((1, OUT_N), lambda i: (0, 0)),
        ],
        out_specs=pl.BlockSpec((BM, OUT_N), lambda i: (i, 0)),
        out_shape=jax.ShapeDtypeStruct((BATCH_N, OUT_N), jnp.float32),
    )(x3, W1r, b1, W2, b2)


def kernel(inputs, table, W1, b1, W2, b2):
    flat_idx = inputs.T.reshape(-1)
    embeds = _gather(table, flat_idx)
    x3 = embeds.reshape(CTX_N, BATCH_N, EMBED)
    return _mlp(x3, W1.reshape(CTX_N, EMBED, HIDDEN),
                b1.reshape(1, HIDDEN), W2, b2.reshape(1, OUT_N))
